# baseline (speedup 1.0000x reference)
import functools

import jax
import jax.numpy as jnp
from jax.experimental import pallas as pl


def _bag_body(idx_ref, tab_ref, out_ref, *, vocab, bag):
    idxT = idx_ref[...].astype(jnp.int32)
    tab = tab_ref[...]
    r = idxT.shape[1]
    iota = jax.lax.broadcasted_iota(jnp.int32, (vocab, r), 0)
    cntT = jnp.zeros((vocab, r), jnp.bfloat16)
    for j in range(bag):
        row = jax.lax.broadcast_in_dim(idxT[j], (vocab, r), (1,))
        cntT = cntT + (row == iota).astype(jnp.bfloat16)
    out_ref[...] = jax.lax.dot_general(
        cntT, tab, (((0,), (0,)), ((), ())),
        preferred_element_type=jnp.float32)


def _bag_call(featsT, table, block):
    bag, n = featsT.shape
    vocab, d = table.shape
    return pl.pallas_call(
        functools.partial(_bag_body, vocab=vocab, bag=bag),
        grid=((n + block - 1) // block,),
        in_specs=[
            pl.BlockSpec((bag, block), lambda i: (0, i)),
            pl.BlockSpec((vocab, d), lambda i: (0, 0)),
        ],
        out_specs=pl.BlockSpec((block, d), lambda i: (i, 0)),
        out_shape=jax.ShapeDtypeStruct((n, d), jnp.float32),
    )(featsT, table.astype(jnp.bfloat16))


def kernel(node_feats, edge_feats, node_table, edge_table):
    node_out = _bag_call(node_feats.astype(jnp.int8).T, node_table, 5120)
    edge_out = _bag_call(edge_feats.astype(jnp.int8).T, edge_table, 12800)
    return node_out, edge_out


# speedup vs baseline: 1.0362x; 1.0362x over previous
import functools

import jax
import jax.numpy as jnp
from jax.experimental import pallas as pl


def _bag_body(idx_ref, tab_ref, out_ref, *, vocab, bag):
    idxT = idx_ref[...]
    tab = tab_ref[...]
    r = idxT.shape[1]
    iota = jax.lax.broadcasted_iota(jnp.int32, (vocab, r), 0)
    cntT = jnp.zeros((vocab, r), jnp.bfloat16)
    for j in range(bag):
        row = jax.lax.broadcast_in_dim(idxT[j], (vocab, r), (1,))
        cntT = cntT + (row == iota).astype(jnp.bfloat16)
    out_ref[...] = jax.lax.dot_general(
        cntT, tab, (((0,), (0,)), ((), ())),
        preferred_element_type=jnp.float32)


def _bag_call(featsT, table, block):
    bag, n = featsT.shape
    vocab, d = table.shape
    return pl.pallas_call(
        functools.partial(_bag_body, vocab=vocab, bag=bag),
        grid=((n + block - 1) // block,),
        in_specs=[
            pl.BlockSpec((bag, block), lambda i: (0, i)),
            pl.BlockSpec((vocab, d), lambda i: (0, 0)),
        ],
        out_specs=pl.BlockSpec((block, d), lambda i: (i, 0)),
        out_shape=jax.ShapeDtypeStruct((n, d), jnp.float32),
    )(featsT, table.astype(jnp.bfloat16))


def kernel(node_feats, edge_feats, node_table, edge_table):
    node_out = _bag_call(node_feats.T, node_table, 5120)
    edge_out = _bag_call(edge_feats.T, edge_table, 12800)
    return node_out, edge_out
